# baseline (speedup 1.0000x reference)
import functools

import jax
import jax.numpy as jnp
from jax import lax
from jax.experimental import pallas as pl
from jax.experimental.pallas import tpu as pltpu
from jax.experimental.pallas import tpu_sc as plsc

B, S, D = 1, 2048, 2048
E, K, DE = 8, 2, 1024
N_SHARED = 2
DS = DE * N_SHARED

TM = 256
NB_R = (S * K) // TM + E
P_R = NB_R * TM
NB = NB_R + N_SHARED * (S // TM)
P = NB * TM

_SC_CORES = 2
_SC_SUBCORES = 16
_NW = _SC_CORES * _SC_SUBCORES


def _gather_rows(table, idx):
    n_rows, d = table.shape
    b = idx.shape[0]
    assert b % (8 * _NW) == 0
    b_per_w = b // _NW
    chunk = 40 if b_per_w % 40 == 0 else 32
    assert b_per_w % chunk == 0 and chunk % 8 == 0
    n_chunks = b_per_w // chunk
    mesh = plsc.VectorSubcoreMesh(core_axis_name="c", subcore_axis_name="s")

    @functools.partial(
        pl.kernel,
        mesh=mesh,
        out_type=jax.ShapeDtypeStruct((b, d), table.dtype),
        scratch_types=[
            pltpu.VMEM((b_per_w,), jnp.int32),
            pltpu.VMEM((chunk, d), table.dtype),
            pltpu.VMEM((chunk, d), table.dtype),
            pltpu.SemaphoreType.DMA,
            pltpu.SemaphoreType.DMA,
        ],
    )
    def k(table_hbm, idx_hbm, out_hbm, idx_v, buf0, buf1, sem0, sem1):
        wid = lax.axis_index("s") * _SC_CORES + lax.axis_index("c")
        base = wid * b_per_w
        bufs, sems = (buf0, buf1), (sem0, sem1)
        pltpu.sync_copy(idx_hbm.at[pl.ds(base, b_per_w)], idx_v)

        def gstart(c):
            return pltpu.async_copy(
                table_hbm.at[idx_v.at[pl.ds(c * chunk, chunk)]],
                bufs[c % 2], sems[c % 2])

        copies = [gstart(0)]
        for c in range(n_chunks):
            if c + 1 < n_chunks:
                copies.append(gstart(c + 1))
            copies[c].wait()
            pltpu.sync_copy(bufs[c % 2],
                            out_hbm.at[pl.ds(base + c * chunk, chunk)])

    return k(table, idx)


def _grouped_ffn_body(gid_ref, xt_ref, wg_ref, wu_ref, wd_ref, w_ref, o_ref):
    x = xt_ref[...]
    g = jnp.dot(x, wg_ref[0], preferred_element_type=jnp.float32)
    u = jnp.dot(x, wu_ref[0], preferred_element_type=jnp.float32)
    h = (g * jax.nn.sigmoid(g) * u).astype(jnp.bfloat16)
    o = jnp.dot(h, wd_ref[0], preferred_element_type=jnp.float32)
    o_ref[...] = (o * w_ref[...]).astype(jnp.bfloat16)


def _grouped_ffn(gid, xt, wg_all, wu_all, wd_all, w2d):
    grid_spec = pltpu.PrefetchScalarGridSpec(
        num_scalar_prefetch=1,
        grid=(NB,),
        in_specs=[
            pl.BlockSpec((TM, D), lambda i, g: (i, 0)),
            pl.BlockSpec((1, D, DE), lambda i, g: (g[i], 0, 0)),
            pl.BlockSpec((1, D, DE), lambda i, g: (g[i], 0, 0)),
            pl.BlockSpec((1, DE, D), lambda i, g: (g[i], 0, 0)),
            pl.BlockSpec((TM, 1), lambda i, g: (i, 0)),
        ],
        out_specs=pl.BlockSpec((TM, D), lambda i, g: (i, 0)),
    )
    return pl.pallas_call(
        _grouped_ffn_body,
        grid_spec=grid_spec,
        out_shape=jax.ShapeDtypeStruct((P, D), jnp.bfloat16),
    )(gid, xt, wg_all, wu_all, wd_all, w2d)


def _combine_body(og_ref, sh1_ref, sh2_ref, o_ref):
    og = og_ref[...].astype(jnp.float32)
    o_ref[...] = (og[:, 0, :] + og[:, 1, :]
                  + sh1_ref[...].astype(jnp.float32)
                  + sh2_ref[...].astype(jnp.float32))


def _combine(og2, o_routed):
    return pl.pallas_call(
        _combine_body,
        grid=(S // TM,),
        in_specs=[
            pl.BlockSpec((TM, K, D), lambda i: (i, 0, 0)),
            pl.BlockSpec((TM, D), lambda i: (P_R // TM + i, 0)),
            pl.BlockSpec((TM, D), lambda i: ((P_R + S) // TM + i, 0)),
        ],
        out_specs=pl.BlockSpec((TM, D), lambda i: (i, 0)),
        out_shape=jax.ShapeDtypeStruct((S, D), jnp.float32),
    )(og2, o_routed, o_routed)


def kernel(x, W_g, W_gate, W_up, W_down, W_gate_s, W_up_s, W_down_s):
    b, s, d = x.shape
    x_flat = x.reshape(-1, d)

    logits = x_flat @ W_g
    scores = jax.nn.softmax(logits, axis=-1)
    topk_scores, topk_idx = jax.lax.top_k(scores, K)

    e_flat = topk_idx.reshape(-1).astype(jnp.int32)
    w_flat = topk_scores.reshape(-1)
    oh = (e_flat[:, None] == jnp.arange(E, dtype=jnp.int32)[None, :]).astype(
        jnp.int32)
    counts = jnp.sum(oh, axis=0)
    nblk = (counts + TM - 1) // TM
    ends_blk = jnp.cumsum(nblk)
    starts_row = (ends_blk - nblk) * TM
    rank = jnp.take_along_axis(jnp.cumsum(oh, axis=0) - oh,
                               e_flat[:, None], axis=1)[:, 0]
    pos_p = starts_row[e_flat] + rank

    tok_ids = (jnp.arange(S * K, dtype=jnp.int32) // K)
    routed_src = jnp.zeros((P_R,), jnp.int32).at[pos_p].set(tok_ids)
    ar_s = jnp.arange(S, dtype=jnp.int32)
    token_src = jnp.concatenate([routed_src, ar_s, ar_s])

    routed_w = jnp.zeros((P_R,), jnp.float32).at[pos_p].set(w_flat)
    w_pad = jnp.concatenate([routed_w, jnp.ones((N_SHARED * S,), jnp.float32)])

    blk_rows = jnp.arange(NB_R, dtype=jnp.int32) * TM
    gid_r = jnp.clip(
        jnp.searchsorted(ends_blk * TM, blk_rows, side="right"), 0, E - 1
    ).astype(jnp.int32)
    gid = jnp.concatenate([
        gid_r,
        jnp.full((S // TM,), E, jnp.int32),
        jnp.full((S // TM,), E + 1, jnp.int32),
    ])


    wg_all = jnp.concatenate(
        [W_gate, W_gate_s.reshape(D, N_SHARED, DE).transpose(1, 0, 2)]
    ).astype(jnp.bfloat16)
    wu_all = jnp.concatenate(
        [W_up, W_up_s.reshape(D, N_SHARED, DE).transpose(1, 0, 2)]
    ).astype(jnp.bfloat16)
    wd_all = jnp.concatenate(
        [W_down, W_down_s.reshape(N_SHARED, DE, D)]
    ).astype(jnp.bfloat16)

    x_pk = lax.bitcast_convert_type(
        x_flat.astype(jnp.bfloat16).reshape(S, D // 2, 2), jnp.float32)
    xt = lax.bitcast_convert_type(
        _gather_rows(x_pk, token_src), jnp.bfloat16).reshape(P, D)
    o_routed = _grouped_ffn(gid, xt, wg_all, wu_all, wd_all, w_pad[:, None])
    o_pk = lax.bitcast_convert_type(
        o_routed.reshape(P, D // 2, 2), jnp.float32)
    og2 = lax.bitcast_convert_type(
        _gather_rows(o_pk, pos_p), jnp.bfloat16).reshape(S, K, D)
    out = _combine(og2, o_routed)
    return out.reshape(b, s, d)


# speedup vs baseline: 5.0733x; 5.0733x over previous
import functools

import jax
import jax.numpy as jnp
from jax import lax
from jax.experimental import pallas as pl
from jax.experimental.pallas import tpu as pltpu
from jax.experimental.pallas import tpu_sc as plsc

B, S, D = 1, 2048, 2048
E, K, DE = 8, 2, 1024
N_SHARED = 2
DS = DE * N_SHARED

TM = 256
NB_R = (S * K) // TM + E
P_R = NB_R * TM
NB = NB_R + N_SHARED * (S // TM)
P = NB * TM

_SC_CORES = 2
_SC_SUBCORES = 16
_NW = _SC_CORES * _SC_SUBCORES


def _gather_rows(table, idx):
    n_rows, d = table.shape
    b = idx.shape[0]
    assert b % (8 * _NW) == 0
    b_per_w = b // _NW
    chunk = 24 if b_per_w % 24 == 0 else 16
    assert b_per_w % chunk == 0 and chunk % 8 == 0
    n_chunks = b_per_w // chunk
    mesh = plsc.VectorSubcoreMesh(core_axis_name="c", subcore_axis_name="s")

    @functools.partial(
        pl.kernel,
        mesh=mesh,
        out_type=jax.ShapeDtypeStruct((b, d), table.dtype),
        scratch_types=[
            pltpu.VMEM((b_per_w,), jnp.int32),
            pltpu.VMEM((chunk, d), table.dtype),
            pltpu.VMEM((chunk, d), table.dtype),
            pltpu.SemaphoreType.DMA,
            pltpu.SemaphoreType.DMA,
        ],
    )
    def k(table_hbm, idx_hbm, out_hbm, idx_v, buf0, buf1, sem0, sem1):
        wid = lax.axis_index("s") * _SC_CORES + lax.axis_index("c")
        base = wid * b_per_w
        bufs, sems = (buf0, buf1), (sem0, sem1)
        pltpu.sync_copy(idx_hbm.at[pl.ds(base, b_per_w)], idx_v)

        def gstart(c):
            return pltpu.async_copy(
                table_hbm.at[idx_v.at[pl.ds(c * chunk, chunk)]],
                bufs[c % 2], sems[c % 2])

        copies = [gstart(0)]
        for c in range(n_chunks):
            if c + 1 < n_chunks:
                copies.append(gstart(c + 1))
            copies[c].wait()
            pltpu.sync_copy(bufs[c % 2],
                            out_hbm.at[pl.ds(base + c * chunk, chunk)])

    return k(table, idx)


def _grouped_ffn_body(gid_ref, xt_ref, x_ref, wg_ref, wu_ref, wd_ref, w_ref,
                      o_ref):
    x = lax.cond(pl.program_id(0) < NB_R,
                 lambda: xt_ref[...], lambda: x_ref[...]).astype(jnp.bfloat16)
    g = jnp.dot(x, wg_ref[0], preferred_element_type=jnp.float32)
    u = jnp.dot(x, wu_ref[0], preferred_element_type=jnp.float32)
    h = (g * jax.nn.sigmoid(g) * u).astype(jnp.bfloat16)
    o = jnp.dot(h, wd_ref[0], preferred_element_type=jnp.float32)
    o_ref[...] = o * w_ref[...]


def _grouped_ffn(gid, xt, x_flat, wg_all, wu_all, wd_all, w2d):
    grid_spec = pltpu.PrefetchScalarGridSpec(
        num_scalar_prefetch=1,
        grid=(NB,),
        in_specs=[
            pl.BlockSpec((TM, D), lambda i, g: (jnp.minimum(i, NB_R - 1), 0)),
            pl.BlockSpec(
                (TM, D),
                lambda i, g: (jnp.where(i < NB_R, 0, (i - NB_R) % (S // TM)),
                              0)),
            pl.BlockSpec((1, D, DE), lambda i, g: (g[i], 0, 0)),
            pl.BlockSpec((1, D, DE), lambda i, g: (g[i], 0, 0)),
            pl.BlockSpec((1, DE, D), lambda i, g: (g[i], 0, 0)),
            pl.BlockSpec((TM, 1), lambda i, g: (i, 0)),
        ],
        out_specs=pl.BlockSpec((TM, D), lambda i, g: (i, 0)),
    )
    return pl.pallas_call(
        _grouped_ffn_body,
        grid_spec=grid_spec,
        out_shape=jax.ShapeDtypeStruct((P, D), jnp.float32),
    )(gid, xt, x_flat, wg_all, wu_all, wd_all, w2d)


def _combine_body(og_ref, sh1_ref, sh2_ref, o_ref):
    og = og_ref[...]
    o_ref[...] = og[:, 0, :] + og[:, 1, :] + sh1_ref[...] + sh2_ref[...]


def _combine(og2, o_routed):
    return pl.pallas_call(
        _combine_body,
        grid=(S // TM,),
        in_specs=[
            pl.BlockSpec((TM, K, D), lambda i: (i, 0, 0)),
            pl.BlockSpec((TM, D), lambda i: (P_R // TM + i, 0)),
            pl.BlockSpec((TM, D), lambda i: ((P_R + S) // TM + i, 0)),
        ],
        out_specs=pl.BlockSpec((TM, D), lambda i: (i, 0)),
        out_shape=jax.ShapeDtypeStruct((S, D), jnp.float32),
    )(og2, o_routed, o_routed)


def kernel(x, W_g, W_gate, W_up, W_down, W_gate_s, W_up_s, W_down_s):
    b, s, d = x.shape
    x_flat = x.reshape(-1, d)

    logits = x_flat @ W_g
    scores = jax.nn.softmax(logits, axis=-1)
    topk_scores, topk_idx = jax.lax.top_k(scores, K)

    e_flat = topk_idx.reshape(-1).astype(jnp.int32)
    w_flat = topk_scores.reshape(-1)
    oh = (e_flat[:, None] == jnp.arange(E, dtype=jnp.int32)[None, :]).astype(
        jnp.int32)
    counts = jnp.sum(oh, axis=0)
    nblk = (counts + TM - 1) // TM
    ends_blk = jnp.cumsum(nblk)
    starts_row = (ends_blk - nblk) * TM
    rank = jnp.take_along_axis(jnp.cumsum(oh, axis=0) - oh,
                               e_flat[:, None], axis=1)[:, 0]
    pos_p = starts_row[e_flat] + rank

    tok_ids = (jnp.arange(S * K, dtype=jnp.int32) // K)
    token_src = jnp.zeros((P_R,), jnp.int32).at[pos_p].set(tok_ids)

    routed_w = jnp.zeros((P_R,), jnp.float32).at[pos_p].set(w_flat)
    w_pad = jnp.concatenate([routed_w, jnp.ones((N_SHARED * S,), jnp.float32)])

    blk_rows = jnp.arange(NB_R, dtype=jnp.int32) * TM
    gid_r = jnp.clip(
        jnp.searchsorted(ends_blk * TM, blk_rows, side="right"), 0, E - 1
    ).astype(jnp.int32)
    gid = jnp.concatenate([
        gid_r,
        jnp.full((S // TM,), E, jnp.int32),
        jnp.full((S // TM,), E + 1, jnp.int32),
    ])


    wg_all = jnp.concatenate(
        [W_gate, W_gate_s.reshape(D, N_SHARED, DE).transpose(1, 0, 2)]
    ).astype(jnp.bfloat16)
    wu_all = jnp.concatenate(
        [W_up, W_up_s.reshape(D, N_SHARED, DE).transpose(1, 0, 2)]
    ).astype(jnp.bfloat16)
    wd_all = jnp.concatenate(
        [W_down, W_down_s.reshape(N_SHARED, DE, D)]
    ).astype(jnp.bfloat16)

    xt = _gather_rows(x_flat, token_src)
    o_routed = _grouped_ffn(gid, xt, x_flat, wg_all, wu_all, wd_all,
                            w_pad[:, None])
    og2 = _gather_rows(o_routed, pos_p)
    out = _combine(og2.reshape(S, K, D), o_routed)
    return out.reshape(b, s, d)


# speedup vs baseline: 7.4726x; 1.4729x over previous
import functools

import jax
import jax.numpy as jnp
from jax import lax
from jax.experimental import pallas as pl
from jax.experimental.pallas import tpu as pltpu
from jax.experimental.pallas import tpu_sc as plsc

B, S, D = 1, 2048, 2048
E, K, DE = 8, 2, 1024
N_SHARED = 2
DS = DE * N_SHARED

TM = 256
NB_R = (S * K) // TM + E
P_R = NB_R * TM
NB = NB_R + N_SHARED * (S // TM)
P = NB * TM

_SC_CORES = 2
_SC_SUBCORES = 16
_NW = _SC_CORES * _SC_SUBCORES


def _gather_rows(table, idx):
    n_rows, d = table.shape
    b = idx.shape[0]
    assert b % (8 * _NW) == 0
    b_per_w = b // _NW
    chunk = 24 if b_per_w % 24 == 0 else 16
    assert b_per_w % chunk == 0 and chunk % 8 == 0
    n_chunks = b_per_w // chunk
    mesh = plsc.VectorSubcoreMesh(core_axis_name="c", subcore_axis_name="s")

    @functools.partial(
        pl.kernel,
        mesh=mesh,
        out_type=jax.ShapeDtypeStruct((b, d), table.dtype),
        scratch_types=[
            pltpu.VMEM((b_per_w,), jnp.int32),
            pltpu.VMEM((chunk, d), table.dtype),
            pltpu.VMEM((chunk, d), table.dtype),
            pltpu.SemaphoreType.DMA,
            pltpu.SemaphoreType.DMA,
        ],
    )
    def k(table_hbm, idx_hbm, out_hbm, idx_v, buf0, buf1, sem0, sem1):
        wid = lax.axis_index("s") * _SC_CORES + lax.axis_index("c")
        base = wid * b_per_w
        bufs, sems = (buf0, buf1), (sem0, sem1)
        pltpu.sync_copy(idx_hbm.at[pl.ds(base, b_per_w)], idx_v)

        def gstart(c):
            return pltpu.async_copy(
                table_hbm.at[idx_v.at[pl.ds(c * chunk, chunk)]],
                bufs[c % 2], sems[c % 2])

        copies = [gstart(0)]
        for c in range(n_chunks):
            if c + 1 < n_chunks:
                copies.append(gstart(c + 1))
            copies[c].wait()
            pltpu.sync_copy(bufs[c % 2],
                            out_hbm.at[pl.ds(base + c * chunk, chunk)])

    return k(table, idx)


TDE = DE // 2


def _routed_ffn_body(gid_ref, xt_ref, wg_ref, wu_ref, wd_ref, w_ref, o_ref):
    x = xt_ref[...].astype(jnp.bfloat16)
    wg = wg_ref[0].astype(jnp.bfloat16)
    wu = wu_ref[0].astype(jnp.bfloat16)
    wd = wd_ref[0].astype(jnp.bfloat16)
    g = jnp.dot(x, wg, preferred_element_type=jnp.float32)
    u = jnp.dot(x, wu, preferred_element_type=jnp.float32)
    h = (g * jax.nn.sigmoid(g) * u).astype(jnp.bfloat16)
    o = jnp.dot(h, wd, preferred_element_type=jnp.float32) * w_ref[...]

    @pl.when(pl.program_id(1) == 0)
    def _():
        o_ref[...] = o

    @pl.when(pl.program_id(1) != 0)
    def _():
        o_ref[...] += o


def _routed_ffn(gid, xt, w_gate, w_up, w_down, w2d):
    grid_spec = pltpu.PrefetchScalarGridSpec(
        num_scalar_prefetch=1,
        grid=(NB_R, DE // TDE),
        in_specs=[
            pl.BlockSpec((TM, D), lambda i, j, g: (i, 0)),
            pl.BlockSpec((1, D, TDE), lambda i, j, g: (g[i], 0, j)),
            pl.BlockSpec((1, D, TDE), lambda i, j, g: (g[i], 0, j)),
            pl.BlockSpec((1, TDE, D), lambda i, j, g: (g[i], j, 0)),
            pl.BlockSpec((TM, 1), lambda i, j, g: (i, 0)),
        ],
        out_specs=pl.BlockSpec((TM, D), lambda i, j, g: (i, 0)),
    )
    return pl.pallas_call(
        _routed_ffn_body,
        grid_spec=grid_spec,
        out_shape=jax.ShapeDtypeStruct((P_R, D), jnp.float32),
    )(gid, xt, w_gate, w_up, w_down, w2d)


def _shared_ffn_body(x_ref, wg_ref, wu_ref, wd_ref, o_ref):
    x = x_ref[...].astype(jnp.bfloat16)
    g = jnp.dot(x, wg_ref[0], preferred_element_type=jnp.float32)
    u = jnp.dot(x, wu_ref[0], preferred_element_type=jnp.float32)
    h = (g * jax.nn.sigmoid(g) * u).astype(jnp.bfloat16)
    o_ref[...] = jnp.dot(h, wd_ref[0], preferred_element_type=jnp.float32)


def _shared_ffn(x_flat, wgs, wus, wds):
    nb_tok = S // TM
    return pl.pallas_call(
        _shared_ffn_body,
        grid=(N_SHARED * nb_tok,),
        in_specs=[
            pl.BlockSpec((TM, D), lambda i: (i % nb_tok, 0)),
            pl.BlockSpec((1, D, DE), lambda i: (i // nb_tok, 0, 0)),
            pl.BlockSpec((1, D, DE), lambda i: (i // nb_tok, 0, 0)),
            pl.BlockSpec((1, DE, D), lambda i: (i // nb_tok, 0, 0)),
        ],
        out_specs=pl.BlockSpec((TM, D), lambda i: (i, 0)),
        out_shape=jax.ShapeDtypeStruct((N_SHARED * S, D), jnp.float32),
    )(x_flat, wgs, wus, wds)


def _combine_body(og_ref, sh1_ref, sh2_ref, o_ref):
    og = og_ref[...]
    o_ref[...] = og[:, 0, :] + og[:, 1, :] + sh1_ref[...] + sh2_ref[...]


def _combine(og2, o_shared):
    return pl.pallas_call(
        _combine_body,
        grid=(S // TM,),
        in_specs=[
            pl.BlockSpec((TM, K, D), lambda i: (i, 0, 0)),
            pl.BlockSpec((TM, D), lambda i: (i, 0)),
            pl.BlockSpec((TM, D), lambda i: (S // TM + i, 0)),
        ],
        out_specs=pl.BlockSpec((TM, D), lambda i: (i, 0)),
        out_shape=jax.ShapeDtypeStruct((S, D), jnp.float32),
    )(og2, o_shared, o_shared)


def kernel(x, W_g, W_gate, W_up, W_down, W_gate_s, W_up_s, W_down_s):
    b, s, d = x.shape
    x_flat = x.reshape(-1, d)

    logits = x_flat @ W_g
    scores = jax.nn.softmax(logits, axis=-1)
    topk_scores, topk_idx = jax.lax.top_k(scores, K)

    e_flat = topk_idx.reshape(-1).astype(jnp.int32)
    w_flat = topk_scores.reshape(-1)
    oh = (e_flat[:, None] == jnp.arange(E, dtype=jnp.int32)[None, :]).astype(
        jnp.int32)
    counts = jnp.sum(oh, axis=0)
    nblk = (counts + TM - 1) // TM
    ends_blk = jnp.cumsum(nblk)
    starts_row = (ends_blk - nblk) * TM
    rank = jnp.take_along_axis(jnp.cumsum(oh, axis=0) - oh,
                               e_flat[:, None], axis=1)[:, 0]
    pos_p = starts_row[e_flat] + rank

    tok_ids = (jnp.arange(S * K, dtype=jnp.int32) // K)
    token_src = (jnp.arange(P_R, dtype=jnp.int32) % S).at[pos_p].set(tok_ids)

    w_pad = jnp.zeros((P_R,), jnp.float32).at[pos_p].set(w_flat)

    blk_rows = jnp.arange(NB_R, dtype=jnp.int32) * TM
    gid = jnp.clip(
        jnp.searchsorted(ends_blk * TM, blk_rows, side="right"), 0, E - 1
    ).astype(jnp.int32)

    wgs = W_gate_s.reshape(D, N_SHARED, DE).transpose(1, 0, 2).astype(
        jnp.bfloat16)
    wus = W_up_s.reshape(D, N_SHARED, DE).transpose(1, 0, 2).astype(
        jnp.bfloat16)
    wds = W_down_s.reshape(N_SHARED, DE, D).astype(jnp.bfloat16)

    xt = _gather_rows(x_flat, token_src)
    o_routed = _routed_ffn(gid, xt, W_gate, W_up, W_down, w_pad[:, None])
    og2 = _gather_rows(o_routed, pos_p)
    o_shared = _shared_ffn(x_flat, wgs, wus, wds)
    out = _combine(og2.reshape(S, K, D), o_shared)
    return out.reshape(b, s, d)


# speedup vs baseline: 8.5094x; 1.1387x over previous
import functools

import jax
import jax.numpy as jnp
from jax import lax
from jax.experimental import pallas as pl
from jax.experimental.pallas import tpu as pltpu
from jax.experimental.pallas import tpu_sc as plsc

B, S, D = 1, 2048, 2048
E, K, DE = 8, 2, 1024
N_SHARED = 2
DS = DE * N_SHARED

TM = 512
NB_R = (S * K) // TM + E
P_R = NB_R * TM
TMS = 256

_SC_CORES = 2
_SC_SUBCORES = 16
_NW = _SC_CORES * _SC_SUBCORES


def _gather_rows(table, idx):
    n_rows, d = table.shape
    b = idx.shape[0]
    assert b % (8 * _NW) == 0
    b_per_w = b // _NW
    chunk = 24 if b_per_w % 24 == 0 else 16
    assert b_per_w % chunk == 0 and chunk % 8 == 0
    n_chunks = b_per_w // chunk
    mesh = plsc.VectorSubcoreMesh(core_axis_name="c", subcore_axis_name="s")

    @functools.partial(
        pl.kernel,
        mesh=mesh,
        out_type=jax.ShapeDtypeStruct((b, d), table.dtype),
        compiler_params=pltpu.CompilerParams(use_tc_tiling_on_sc=True),
        scratch_types=[
            pltpu.VMEM((b_per_w,), jnp.int32),
            pltpu.VMEM((chunk, d), table.dtype),
            pltpu.VMEM((chunk, d), table.dtype),
            pltpu.SemaphoreType.DMA,
            pltpu.SemaphoreType.DMA,
        ],
    )
    def k(table_hbm, idx_hbm, out_hbm, idx_v, buf0, buf1, sem0, sem1):
        wid = lax.axis_index("s") * _SC_CORES + lax.axis_index("c")
        base = wid * b_per_w
        bufs, sems = (buf0, buf1), (sem0, sem1)
        pltpu.sync_copy(idx_hbm.at[pl.ds(base, b_per_w)], idx_v)

        def gstart(c):
            return pltpu.async_copy(
                table_hbm.at[idx_v.at[pl.ds(c * chunk, chunk)]],
                bufs[c % 2], sems[c % 2])

        copies = [gstart(0)]
        for c in range(n_chunks):
            if c + 1 < n_chunks:
                copies.append(gstart(c + 1))
            copies[c].wait()
            pltpu.sync_copy(bufs[c % 2],
                            out_hbm.at[pl.ds(base + c * chunk, chunk)])

    return k(table, idx)


TDE = DE // 2


def _routed_ffn_body(gid_ref, xt_ref, wg_ref, wu_ref, wd_ref, w_ref, o_ref):
    x = xt_ref[...].astype(jnp.bfloat16)
    wg = wg_ref[0].astype(jnp.bfloat16)
    wu = wu_ref[0].astype(jnp.bfloat16)
    wd = wd_ref[0].astype(jnp.bfloat16)
    g = jnp.dot(x, wg, preferred_element_type=jnp.float32)
    u = jnp.dot(x, wu, preferred_element_type=jnp.float32)
    h = (g * jax.nn.sigmoid(g) * u).astype(jnp.bfloat16)
    o = jnp.dot(h, wd, preferred_element_type=jnp.float32) * w_ref[...]

    @pl.when(pl.program_id(1) == 0)
    def _():
        o_ref[...] = o

    @pl.when(pl.program_id(1) != 0)
    def _():
        o_ref[...] += o


def _routed_ffn(gid, xt, w_gate, w_up, w_down, w2d):
    grid_spec = pltpu.PrefetchScalarGridSpec(
        num_scalar_prefetch=1,
        grid=(NB_R, DE // TDE),
        in_specs=[
            pl.BlockSpec((TM, D), lambda i, j, g: (i, 0)),
            pl.BlockSpec((1, D, TDE), lambda i, j, g: (g[i], 0, j)),
            pl.BlockSpec((1, D, TDE), lambda i, j, g: (g[i], 0, j)),
            pl.BlockSpec((1, TDE, D), lambda i, j, g: (g[i], j, 0)),
            pl.BlockSpec((TM, 1), lambda i, j, g: (i, 0)),
        ],
        out_specs=pl.BlockSpec((TM, D), lambda i, j, g: (i, 0)),
    )
    return pl.pallas_call(
        _routed_ffn_body,
        grid_spec=grid_spec,
        out_shape=jax.ShapeDtypeStruct((P_R, D), jnp.float32),
    )(gid, xt, w_gate, w_up, w_down, w2d)


def _shared_ffn_body(x_ref, wg_ref, wu_ref, wd_ref, o_ref):
    x = x_ref[...].astype(jnp.bfloat16)
    g = jnp.dot(x, wg_ref[0], preferred_element_type=jnp.float32)
    u = jnp.dot(x, wu_ref[0], preferred_element_type=jnp.float32)
    h = (g * jax.nn.sigmoid(g) * u).astype(jnp.bfloat16)
    o_ref[...] = jnp.dot(h, wd_ref[0], preferred_element_type=jnp.float32)


def _shared_ffn(x_flat, wgs, wus, wds):
    nb_tok = S // TMS
    return pl.pallas_call(
        _shared_ffn_body,
        grid=(N_SHARED * nb_tok,),
        in_specs=[
            pl.BlockSpec((TMS, D), lambda i: (i % nb_tok, 0)),
            pl.BlockSpec((1, D, DE), lambda i: (i // nb_tok, 0, 0)),
            pl.BlockSpec((1, D, DE), lambda i: (i // nb_tok, 0, 0)),
            pl.BlockSpec((1, DE, D), lambda i: (i // nb_tok, 0, 0)),
        ],
        out_specs=pl.BlockSpec((TMS, D), lambda i: (i, 0)),
        out_shape=jax.ShapeDtypeStruct((N_SHARED * S, D), jnp.float32),
    )(x_flat, wgs, wus, wds)


def _combine_body(og0_ref, og1_ref, sh1_ref, sh2_ref, o_ref):
    o_ref[...] = (og0_ref[...] + og1_ref[...]
                  + sh1_ref[...] + sh2_ref[...])


def _combine(og2, o_shared):
    nb_tok = S // TMS
    return pl.pallas_call(
        _combine_body,
        grid=(nb_tok,),
        in_specs=[
            pl.BlockSpec((TMS, D), lambda i: (i, 0)),
            pl.BlockSpec((TMS, D), lambda i: (nb_tok + i, 0)),
            pl.BlockSpec((TMS, D), lambda i: (i, 0)),
            pl.BlockSpec((TMS, D), lambda i: (nb_tok + i, 0)),
        ],
        out_specs=pl.BlockSpec((TMS, D), lambda i: (i, 0)),
        out_shape=jax.ShapeDtypeStruct((S, D), jnp.float32),
    )(og2, og2, o_shared, o_shared)


def kernel(x, W_g, W_gate, W_up, W_down, W_gate_s, W_up_s, W_down_s):
    b, s, d = x.shape
    x_flat = x.reshape(-1, d)

    logits = x_flat @ W_g
    scores = jax.nn.softmax(logits, axis=-1)
    topk_scores, topk_idx = jax.lax.top_k(scores, K)

    e_flat = topk_idx.reshape(-1).astype(jnp.int32)
    w_flat = topk_scores.reshape(-1)
    oh = (e_flat[:, None] == jnp.arange(E, dtype=jnp.int32)[None, :]).astype(
        jnp.int32)
    counts = jnp.sum(oh, axis=0)
    nblk = (counts + TM - 1) // TM
    ends_blk = jnp.cumsum(nblk)
    starts_row = (ends_blk - nblk) * TM
    rank = jnp.take_along_axis(jnp.cumsum(oh, axis=0) - oh,
                               e_flat[:, None], axis=1)[:, 0]
    pos_p = starts_row[e_flat] + rank

    tok_ids = (jnp.arange(S * K, dtype=jnp.int32) // K)
    token_src = (jnp.arange(P_R, dtype=jnp.int32) % S).at[pos_p].set(
        tok_ids, unique_indices=True)

    w_pad = jnp.zeros((P_R,), jnp.float32).at[pos_p].set(
        w_flat, unique_indices=True)

    blk_rows = jnp.arange(NB_R, dtype=jnp.int32) * TM
    gid = jnp.clip(
        jnp.searchsorted(ends_blk * TM, blk_rows, side="right"), 0, E - 1
    ).astype(jnp.int32)

    wgs = W_gate_s.reshape(D, N_SHARED, DE).transpose(1, 0, 2).astype(
        jnp.bfloat16)
    wus = W_up_s.reshape(D, N_SHARED, DE).transpose(1, 0, 2).astype(
        jnp.bfloat16)
    wds = W_down_s.reshape(N_SHARED, DE, D).astype(jnp.bfloat16)

    pos_km = pos_p.reshape(S, K).T.reshape(-1)
    xt = _gather_rows(x_flat, token_src)
    o_routed = _routed_ffn(gid, xt, W_gate, W_up, W_down, w_pad[:, None])
    og2 = _gather_rows(o_routed, pos_km)
    o_shared = _shared_ffn(x_flat, wgs, wus, wds)
    out = _combine(og2, o_shared)
    return out.reshape(b, s, d)


# speedup vs baseline: 8.7091x; 1.0235x over previous
import functools

import jax
import jax.numpy as jnp
from jax import lax
from jax.experimental import pallas as pl
from jax.experimental.pallas import tpu as pltpu
from jax.experimental.pallas import tpu_sc as plsc

B, S, D = 1, 2048, 2048
E, K, DE = 8, 2, 1024
N_SHARED = 2
DS = DE * N_SHARED

TM = 512
NB_R = (S * K) // TM + E
P_R = NB_R * TM
TMS = 256

_SC_CORES = 2
_SC_SUBCORES = 16
_NW = _SC_CORES * _SC_SUBCORES


def _gather_rows(table, idx):
    n_rows, d = table.shape
    b = idx.shape[0]
    assert b % (8 * _NW) == 0
    b_per_w = b // _NW
    chunk = 24 if b_per_w % 24 == 0 else 16
    assert b_per_w % chunk == 0 and chunk % 8 == 0
    n_chunks = b_per_w // chunk
    mesh = plsc.VectorSubcoreMesh(core_axis_name="c", subcore_axis_name="s")

    @functools.partial(
        pl.kernel,
        mesh=mesh,
        out_type=jax.ShapeDtypeStruct((b, d), table.dtype),
        compiler_params=pltpu.CompilerParams(use_tc_tiling_on_sc=True),
        scratch_types=[
            pltpu.VMEM((b_per_w,), jnp.int32),
            pltpu.VMEM((chunk, d), table.dtype),
            pltpu.VMEM((chunk, d), table.dtype),
            pltpu.SemaphoreType.DMA,
            pltpu.SemaphoreType.DMA,
        ],
    )
    def k(table_hbm, idx_hbm, out_hbm, idx_v, buf0, buf1, sem0, sem1):
        wid = lax.axis_index("s") * _SC_CORES + lax.axis_index("c")
        base = wid * b_per_w
        bufs, sems = (buf0, buf1), (sem0, sem1)
        pltpu.sync_copy(idx_hbm.at[pl.ds(base, b_per_w)], idx_v)

        def gstart(c):
            return pltpu.async_copy(
                table_hbm.at[idx_v.at[pl.ds(c * chunk, chunk)]],
                bufs[c % 2], sems[c % 2])

        copies = [gstart(0)]
        for c in range(n_chunks):
            if c + 1 < n_chunks:
                copies.append(gstart(c + 1))
            copies[c].wait()
            pltpu.sync_copy(bufs[c % 2],
                            out_hbm.at[pl.ds(base + c * chunk, chunk)])

    return k(table, idx)


TDE = DE // 2


def _routed_ffn_body(gid_ref, xt_ref, wg_ref, wu_ref, wd_ref, w_ref, o_ref):
    @pl.when(pl.program_id(0) < gid_ref[NB_R])
    def _():
        x = xt_ref[...].astype(jnp.bfloat16)
        wg = wg_ref[0].astype(jnp.bfloat16)
        wu = wu_ref[0].astype(jnp.bfloat16)
        wd = wd_ref[0].astype(jnp.bfloat16)
        g = jnp.dot(x, wg, preferred_element_type=jnp.float32)
        u = jnp.dot(x, wu, preferred_element_type=jnp.float32)
        h = (g * jax.nn.sigmoid(g) * u).astype(jnp.bfloat16)
        o = jnp.dot(h, wd, preferred_element_type=jnp.float32) * w_ref[...]

        @pl.when(pl.program_id(1) == 0)
        def _():
            o_ref[...] = o

        @pl.when(pl.program_id(1) != 0)
        def _():
            o_ref[...] += o


def _routed_ffn(gid, xt, w_gate, w_up, w_down, w2d):
    grid_spec = pltpu.PrefetchScalarGridSpec(
        num_scalar_prefetch=1,
        grid=(NB_R, DE // TDE),
        in_specs=[
            pl.BlockSpec((TM, D),
                         lambda i, j, g: (jnp.minimum(i, g[NB_R] - 1), 0)),
            pl.BlockSpec((1, D, TDE), lambda i, j, g: (g[i], 0, j)),
            pl.BlockSpec((1, D, TDE), lambda i, j, g: (g[i], 0, j)),
            pl.BlockSpec((1, TDE, D), lambda i, j, g: (g[i], j, 0)),
            pl.BlockSpec((TM, 1),
                         lambda i, j, g: (jnp.minimum(i, g[NB_R] - 1), 0)),
        ],
        out_specs=pl.BlockSpec((TM, D), lambda i, j, g: (i, 0)),
    )
    return pl.pallas_call(
        _routed_ffn_body,
        grid_spec=grid_spec,
        out_shape=jax.ShapeDtypeStruct((P_R, D), jnp.float32),
    )(gid, xt, w_gate, w_up, w_down, w2d)


def _shared_ffn_body(x_ref, wg_ref, wu_ref, wd_ref, o_ref):
    x = x_ref[...].astype(jnp.bfloat16)
    g = jnp.dot(x, wg_ref[0], preferred_element_type=jnp.float32)
    u = jnp.dot(x, wu_ref[0], preferred_element_type=jnp.float32)
    h = (g * jax.nn.sigmoid(g) * u).astype(jnp.bfloat16)
    o_ref[...] = jnp.dot(h, wd_ref[0], preferred_element_type=jnp.float32)


def _shared_ffn(x_flat, wgs, wus, wds):
    nb_tok = S // TMS
    return pl.pallas_call(
        _shared_ffn_body,
        grid=(N_SHARED * nb_tok,),
        in_specs=[
            pl.BlockSpec((TMS, D), lambda i: (i % nb_tok, 0)),
            pl.BlockSpec((1, D, DE), lambda i: (i // nb_tok, 0, 0)),
            pl.BlockSpec((1, D, DE), lambda i: (i // nb_tok, 0, 0)),
            pl.BlockSpec((1, DE, D), lambda i: (i // nb_tok, 0, 0)),
        ],
        out_specs=pl.BlockSpec((TMS, D), lambda i: (i, 0)),
        out_shape=jax.ShapeDtypeStruct((N_SHARED * S, D), jnp.float32),
    )(x_flat, wgs, wus, wds)


def _combine_body(og0_ref, og1_ref, sh1_ref, sh2_ref, o_ref):
    o_ref[...] = (og0_ref[...] + og1_ref[...]
                  + sh1_ref[...] + sh2_ref[...])


def _combine(og2, o_shared):
    nb_tok = S // TMS
    return pl.pallas_call(
        _combine_body,
        grid=(nb_tok,),
        in_specs=[
            pl.BlockSpec((TMS, D), lambda i: (i, 0)),
            pl.BlockSpec((TMS, D), lambda i: (nb_tok + i, 0)),
            pl.BlockSpec((TMS, D), lambda i: (i, 0)),
            pl.BlockSpec((TMS, D), lambda i: (nb_tok + i, 0)),
        ],
        out_specs=pl.BlockSpec((TMS, D), lambda i: (i, 0)),
        out_shape=jax.ShapeDtypeStruct((S, D), jnp.float32),
    )(og2, og2, o_shared, o_shared)


def kernel(x, W_g, W_gate, W_up, W_down, W_gate_s, W_up_s, W_down_s):
    b, s, d = x.shape
    x_flat = x.reshape(-1, d)

    logits = x_flat @ W_g
    scores = jax.nn.softmax(logits, axis=-1)
    topk_scores, topk_idx = jax.lax.top_k(scores, K)

    e_flat = topk_idx.reshape(-1).astype(jnp.int32)
    w_flat = topk_scores.reshape(-1)
    oh = (e_flat[:, None] == jnp.arange(E, dtype=jnp.int32)[None, :]).astype(
        jnp.int32)
    counts = jnp.sum(oh, axis=0)
    nblk = (counts + TM - 1) // TM
    ends_blk = jnp.cumsum(nblk)
    starts_row = (ends_blk - nblk) * TM
    rank = jnp.take_along_axis(jnp.cumsum(oh, axis=0) - oh,
                               e_flat[:, None], axis=1)[:, 0]
    pos_p = starts_row[e_flat] + rank

    tok_ids = (jnp.arange(S * K, dtype=jnp.int32) // K)
    md0 = jnp.stack(
        [(jnp.arange(P_R, dtype=jnp.int32) % S).astype(jnp.float32),
         jnp.zeros((P_R,), jnp.float32)], axis=1)
    md = md0.at[pos_p].set(
        jnp.stack([tok_ids.astype(jnp.float32), w_flat], axis=1),
        unique_indices=True)
    token_src = md[:, 0].astype(jnp.int32)
    w_pad = md[:, 1]

    n_blk_used = ends_blk[E - 1].astype(jnp.int32)
    blk_rows = jnp.arange(NB_R, dtype=jnp.int32) * TM
    gid = jnp.clip(
        jnp.searchsorted(ends_blk * TM, blk_rows, side="right"), 0, E - 1
    ).astype(jnp.int32)
    last_gid = jnp.take(gid, jnp.maximum(n_blk_used - 1, 0))
    gid = jnp.where(jnp.arange(NB_R) < n_blk_used, gid, last_gid)
    gid = jnp.concatenate([gid, n_blk_used[None]])

    wgs = W_gate_s.reshape(D, N_SHARED, DE).transpose(1, 0, 2).astype(
        jnp.bfloat16)
    wus = W_up_s.reshape(D, N_SHARED, DE).transpose(1, 0, 2).astype(
        jnp.bfloat16)
    wds = W_down_s.reshape(N_SHARED, DE, D).astype(jnp.bfloat16)

    pos_km = pos_p.reshape(S, K).T.reshape(-1)
    xt = _gather_rows(x_flat, token_src)
    o_routed = _routed_ffn(gid, xt, W_gate, W_up, W_down, w_pad[:, None])
    og2 = _gather_rows(o_routed, pos_km)
    o_shared = _shared_ffn(x_flat, wgs, wus, wds)
    out = _combine(og2, o_shared)
    return out.reshape(b, s, d)
